# baseline (speedup 1.0000x reference)
import functools

import jax
import jax.numpy as jnp
from jax import lax
from jax.experimental import pallas as pl
from jax.experimental.pallas import tpu as pltpu
from jax.experimental.pallas import tpu_sc as plsc

B = 16384
D = 32
NC = 2
NS = 16
NW = NC * NS
BPW = B // NW
CHUNK = 128
NCH = BPW // CHUNK

_mesh = plsc.VectorSubcoreMesh(core_axis_name="c", subcore_axis_name="s")
_params = pltpu.CompilerParams(
    needs_layout_passes=False, use_tc_tiling_on_sc=False)


@functools.partial(
    pl.kernel,
    mesh=_mesh,
    out_type=[jax.ShapeDtypeStruct((B, D), jnp.float32),
              jax.ShapeDtypeStruct((B,), jnp.float32)],
    compiler_params=_params,
    scratch_types=[
        pltpu.VMEM((NCH, CHUNK), jnp.int32),
        pltpu.VMEM((BPW, D), jnp.float32),
        pltpu.VMEM((BPW,), jnp.float32),
        pltpu.SemaphoreType.DMA,
    ],
)
def _gather_one(idx_hbm, table_hbm, bias_hbm, rows_out, bias_out,
                idx_v, rows_v, bias_v, sem):
    wid = lax.axis_index("s") * NC + lax.axis_index("c")
    base = wid * BPW

    pltpu.sync_copy(idx_hbm.at[pl.ds(wid * NCH, NCH)], idx_v)
    copies = []
    for j in range(NCH):
        dst = pl.ds(j * CHUNK, CHUNK)
        copies.append(pltpu.async_copy(
            table_hbm.at[idx_v.at[j]], rows_v.at[dst], sem))
        copies.append(pltpu.async_copy(
            bias_hbm.at[idx_v.at[j]], bias_v.at[dst], sem))
    for c in copies:
        c.wait()

    pltpu.sync_copy(rows_v, rows_out.at[pl.ds(base, BPW)])
    pltpu.sync_copy(bias_v, bias_out.at[pl.ds(base, BPW)])


@functools.partial(
    pl.kernel,
    mesh=_mesh,
    out_type=jax.ShapeDtypeStruct((B,), jnp.float32),
    compiler_params=_params,
    scratch_types=[
        pltpu.VMEM((BPW, D), jnp.float32),
        pltpu.VMEM((BPW, D), jnp.float32),
        pltpu.VMEM((BPW,), jnp.float32),
        pltpu.VMEM((BPW,), jnp.float32),
        pltpu.VMEM((BPW,), jnp.float32),
    ],
)
def _combine(u_hbm, v_hbm, bu_hbm, bi_hbm, out_hbm,
             u_v, v_v, bu_v, bi_v, out_v):
    wid = lax.axis_index("s") * NC + lax.axis_index("c")
    base = wid * BPW

    pltpu.sync_copy(u_hbm.at[pl.ds(base, BPW)], u_v)
    pltpu.sync_copy(v_hbm.at[pl.ds(base, BPW)], v_v)
    pltpu.sync_copy(bu_hbm.at[pl.ds(base, BPW)], bu_v)
    pltpu.sync_copy(bi_hbm.at[pl.ds(base, BPW)], bi_v)

    def body(i, carry):
        r0 = i * 16
        rows = r0 + lax.iota(jnp.int32, 16)
        acc = bu_v[pl.ds(r0, 16)] + bi_v[pl.ds(r0, 16)]
        for d in range(D):
            dd = jnp.full((16,), d, jnp.int32)
            acc = acc + (plsc.load_gather(u_v, [rows, dd])
                         * plsc.load_gather(v_v, [rows, dd]))
        out_v[pl.ds(r0, 16)] = acc
        return carry

    lax.fori_loop(0, BPW // 16, body, 0)
    pltpu.sync_copy(out_v, out_hbm.at[pl.ds(base, BPW)])


def kernel(users, items, user_emb, item_emb, user_bias, item_bias):
    users2 = users.astype(jnp.int32).reshape(B // CHUNK, CHUNK)
    items2 = items.astype(jnp.int32).reshape(B // CHUNK, CHUNK)
    u_rows, bu = _gather_one(users2, user_emb, user_bias.reshape(-1))
    v_rows, bi = _gather_one(items2, item_emb, item_bias.reshape(-1))
    return _combine(u_rows, v_rows, bu, bi)
